# baseline (speedup 1.0000x reference)
import functools

import jax
import jax.numpy as jnp
from jax import lax
from jax.experimental import pallas as pl
from jax.experimental.pallas import tpu as pltpu
from jax.experimental.pallas import tpu_sc as plsc

N = 10000
M = 256
K = 16
OUT = 512

NC, NS = 2, 16
NW = NC * NS
DV = M // 16


C3 = 8

FAST_CORE = 0
MP = 128
GP = 8
TCN = 5000
TROWS = 200
FW, SW = 256, 64
FAST_TOT = NS * FW
N_PAD_SC = NS * (FW + SW)
N_PAD = TCN + N_PAD_SC


NBUF = 2


def _sc_body(x_hbm, idxf_hbm, h_hbm,
             idx_all, buf0, buf1, out_v,
             gsem0, gsem1, wsem):
    c = lax.axis_index("c")
    s = lax.axis_index("s")
    is_fast = c == FAST_CORE
    base = jnp.where(is_fast, s * FW, FAST_TOT + s * SW)
    nchunk = jnp.where(is_fast, FW // C3, SW // C3)
    nrounds = nchunk // NBUF
    bufs = (buf0, buf1)
    gsems = (gsem0, gsem1)

    @pl.when(is_fast)
    def _():
        pltpu.sync_copy(idxf_hbm.at[pl.ds(base * K, FW * K)], idx_all)

    @pl.when(jnp.logical_not(is_fast))
    def _():
        pltpu.sync_copy(idxf_hbm.at[pl.ds(base * K, SW * K)],
                        idx_all.at[pl.ds(0, SW * K)])

    def gdesc(chunk, b):
        return pltpu.make_async_copy(
            x_hbm.at[idx_all.at[pl.ds(chunk * (C3 * K), C3 * K)]],
            bufs[b], gsems[b])

    for bb in range(NBUF):
        gdesc(bb, bb).start()

    @pl.loop(0, nrounds)
    def _round(g):
        @pl.when(g > 0)
        def _():
            pltpu.make_async_copy(out_v, h_hbm.at[pl.ds(0, NBUF * C3)],
                                  wsem).wait()
        for b in range(NBUF):
            chunk = NBUF * g + b
            buf = bufs[b]
            gdesc(chunk, b).wait()

            @plsc.parallel_loop(0, C3 * DV)
            def _red(i):
                cc = i // DV
                sl = pl.ds((i % DV) * 16, 16)
                rb = cc * K
                vals = [buf[rb + k, sl] for k in range(K)]
                while len(vals) > 1:
                    vals = [vals[j] + vals[j + 1]
                            for j in range(0, len(vals) - 1, 2)] + (
                                [vals[-1]] if len(vals) % 2 else [])
                out_v[b * C3 + cc, sl] = vals[0]

            @pl.when(chunk + NBUF < nchunk)
            def _():
                gdesc(chunk + NBUF, b).start()

        pltpu.async_copy(
            out_v, h_hbm.at[pl.ds(base + g * (NBUF * C3), NBUF * C3)], wsem)

    pltpu.make_async_copy(out_v, h_hbm.at[pl.ds(0, NBUF * C3)], wsem).wait()


def _neighbor_sum(x, idxf_pad):
    kfn = pl.kernel(
        _sc_body,
        out_type=jax.ShapeDtypeStruct((N_PAD_SC, M), jnp.float32),
        mesh=plsc.VectorSubcoreMesh(core_axis_name="c", subcore_axis_name="s"),
        scratch_types=[
            pltpu.VMEM((FW * K,), jnp.int32),
            pltpu.VMEM((C3 * K, M), jnp.float32),
            pltpu.VMEM((C3 * K, M), jnp.float32),
            pltpu.VMEM((NBUF * C3, M), jnp.float32),
            pltpu.SemaphoreType.DMA,
            pltpu.SemaphoreType.DMA,
            pltpu.SemaphoreType.DMA,
        ],
    )
    return kfn(x, idxf_pad)


ROWS = 1000
GRID = N // ROWS


def _tc_gather_body(idx_ref, x_ref, o_ref):
    def body(n, carry):
        acc = x_ref[pl.ds(idx_ref[n, 0], 1), :]
        for k in range(1, K):
            acc = acc + x_ref[pl.ds(idx_ref[n, k], 1), :]
        o_ref[pl.ds(n, 1), :] = acc
        return carry

    lax.fori_loop(0, TROWS, body, 0)


def _tc_gather(x, idx_tc):
    return pl.pallas_call(
        _tc_gather_body,
        grid=(TCN // TROWS,),
        in_specs=[
            pl.BlockSpec((TROWS, K), lambda i: (i, 0),
                         memory_space=pltpu.SMEM),
            pl.BlockSpec((N, M), lambda i: (0, 0)),
        ],
        out_specs=pl.BlockSpec((TROWS, M), lambda i: (i, 0)),
        out_shape=jax.ShapeDtypeStruct((TCN, M), jnp.float32),
    )(idx_tc, x)


def _tc_matmul_body(x_ref, hnb_ref, wt_ref, b_ref, y_ref, s_ref, s2_ref):
    i = pl.program_id(0)
    h = x_ref[...] + hnb_ref[...]
    y = jnp.dot(h, wt_ref[...], preferred_element_type=jnp.float32) + b_ref[...]
    y_ref[...] = y
    s = jnp.sum(y, axis=0, keepdims=True)
    s2 = jnp.sum(y * y, axis=0, keepdims=True)

    @pl.when(i == 0)
    def _():
        s_ref[...] = s
        s2_ref[...] = s2

    @pl.when(i > 0)
    def _():
        s_ref[...] += s
        s2_ref[...] += s2


def _tc_bn_body(y_ref, s_ref, s2_ref, g_ref, beta_ref, o_ref):
    mean = s_ref[...] * (1.0 / N)
    var = s2_ref[...] * (1.0 / N) - mean * mean
    scale = g_ref[...] * lax.rsqrt(var + 1e-5)
    shift = beta_ref[...] - mean * scale
    o_ref[...] = jnp.maximum(y_ref[...] * scale + shift, 0.0)


def kernel(nodes_features, nodes_neighbors_indexes, W, b, gamma, beta):
    x = nodes_features
    idxf_pad = jnp.pad(nodes_neighbors_indexes.reshape(-1),
                       (0, (N_PAD - N) * K))
    hnb_sc = _neighbor_sum(x, idxf_pad[TCN * K:])
    hnb_tc = _tc_gather(x, nodes_neighbors_indexes[:TCN])
    hnb = jnp.concatenate([hnb_tc, hnb_sc], axis=0)

    wt = W.T
    b2 = b.reshape(1, OUT)
    g2 = gamma.reshape(1, OUT)
    beta2 = beta.reshape(1, OUT)

    y, s, s2 = pl.pallas_call(
        _tc_matmul_body,
        grid=(GRID,),
        in_specs=[
            pl.BlockSpec((ROWS, M), lambda i: (i, 0)),
            pl.BlockSpec((ROWS, M), lambda i: (i, 0)),
            pl.BlockSpec((M, OUT), lambda i: (0, 0)),
            pl.BlockSpec((1, OUT), lambda i: (0, 0)),
        ],
        out_specs=[
            pl.BlockSpec((ROWS, OUT), lambda i: (i, 0)),
            pl.BlockSpec((1, OUT), lambda i: (0, 0)),
            pl.BlockSpec((1, OUT), lambda i: (0, 0)),
        ],
        out_shape=[
            jax.ShapeDtypeStruct((N, OUT), jnp.float32),
            jax.ShapeDtypeStruct((1, OUT), jnp.float32),
            jax.ShapeDtypeStruct((1, OUT), jnp.float32),
        ],
    )(x, hnb, wt, b2)

    out = pl.pallas_call(
        _tc_bn_body,
        grid=(GRID,),
        in_specs=[
            pl.BlockSpec((ROWS, OUT), lambda i: (i, 0)),
            pl.BlockSpec((1, OUT), lambda i: (0, 0)),
            pl.BlockSpec((1, OUT), lambda i: (0, 0)),
            pl.BlockSpec((1, OUT), lambda i: (0, 0)),
            pl.BlockSpec((1, OUT), lambda i: (0, 0)),
        ],
        out_specs=pl.BlockSpec((ROWS, OUT), lambda i: (i, 0)),
        out_shape=jax.ShapeDtypeStruct((N, OUT), jnp.float32),
    )(y, s, s2, g2, beta2)

    return (out, nodes_neighbors_indexes)


# speedup vs baseline: 1.0999x; 1.0999x over previous
import functools

import jax
import jax.numpy as jnp
from jax import lax
from jax.experimental import pallas as pl
from jax.experimental.pallas import tpu as pltpu
from jax.experimental.pallas import tpu_sc as plsc

N = 10000
M = 256
K = 16
OUT = 512

NC, NS = 2, 16
NW = NC * NS
DV = M // 16


C3 = 8

FAST_CORE = 0
MP = 128
GP = 8
TCN = 4400
TROWS = 400
FW, SW = 288, 64
FAST_TOT = NS * FW
N_PAD_SC = NS * (FW + SW)
N_PAD = TCN + N_PAD_SC


NBUF = 2


def _sc_body(x_hbm, idxf_hbm, h_hbm,
             idx_all, buf0, buf1, out_v,
             gsem0, gsem1, wsem):
    c = lax.axis_index("c")
    s = lax.axis_index("s")
    is_fast = c == FAST_CORE
    base = jnp.where(is_fast, s * FW, FAST_TOT + s * SW)
    nchunk = jnp.where(is_fast, FW // C3, SW // C3)
    nrounds = nchunk // NBUF
    bufs = (buf0, buf1)
    gsems = (gsem0, gsem1)

    @pl.when(is_fast)
    def _():
        pltpu.sync_copy(idxf_hbm.at[pl.ds(base * K, FW * K)], idx_all)

    @pl.when(jnp.logical_not(is_fast))
    def _():
        pltpu.sync_copy(idxf_hbm.at[pl.ds(base * K, SW * K)],
                        idx_all.at[pl.ds(0, SW * K)])

    def gdesc(chunk, b):
        return pltpu.make_async_copy(
            x_hbm.at[idx_all.at[pl.ds(chunk * (C3 * K), C3 * K)]],
            bufs[b], gsems[b])

    for bb in range(NBUF):
        gdesc(bb, bb).start()

    @pl.loop(0, nrounds)
    def _round(g):
        @pl.when(g > 0)
        def _():
            pltpu.make_async_copy(out_v, h_hbm.at[pl.ds(0, NBUF * C3)],
                                  wsem).wait()
        for b in range(NBUF):
            chunk = NBUF * g + b
            buf = bufs[b]
            gdesc(chunk, b).wait()

            @plsc.parallel_loop(0, C3 * DV)
            def _red(i):
                cc = i // DV
                sl = pl.ds((i % DV) * 16, 16)
                rb = cc * K
                vals = [buf[rb + k, sl] for k in range(K)]
                while len(vals) > 1:
                    vals = [vals[j] + vals[j + 1]
                            for j in range(0, len(vals) - 1, 2)] + (
                                [vals[-1]] if len(vals) % 2 else [])
                out_v[b * C3 + cc, sl] = vals[0]

            @pl.when(chunk + NBUF < nchunk)
            def _():
                gdesc(chunk + NBUF, b).start()

        pltpu.async_copy(
            out_v, h_hbm.at[pl.ds(base + g * (NBUF * C3), NBUF * C3)], wsem)

    pltpu.make_async_copy(out_v, h_hbm.at[pl.ds(0, NBUF * C3)], wsem).wait()


def _neighbor_sum(x, idxf_pad):
    kfn = pl.kernel(
        _sc_body,
        out_type=jax.ShapeDtypeStruct((N_PAD_SC, M), jnp.float32),
        mesh=plsc.VectorSubcoreMesh(core_axis_name="c", subcore_axis_name="s"),
        scratch_types=[
            pltpu.VMEM((FW * K,), jnp.int32),
            pltpu.VMEM((C3 * K, M), jnp.float32),
            pltpu.VMEM((C3 * K, M), jnp.float32),
            pltpu.VMEM((NBUF * C3, M), jnp.float32),
            pltpu.SemaphoreType.DMA,
            pltpu.SemaphoreType.DMA,
            pltpu.SemaphoreType.DMA,
        ],
    )
    return kfn(x, idxf_pad)


ROWS = 1000
GRID = N // ROWS


def _tc_gather_body(idx_ref, x_ref, o_ref):
    def body(n, carry):
        acc = x_ref[pl.ds(idx_ref[n, 0], 1), :]
        for k in range(1, K):
            acc = acc + x_ref[pl.ds(idx_ref[n, k], 1), :]
        o_ref[pl.ds(n, 1), :] = acc
        return carry

    lax.fori_loop(0, TROWS, body, 0)


def _tc_gather(x, idx_tc):
    return pl.pallas_call(
        _tc_gather_body,
        grid=(TCN // TROWS,),
        in_specs=[
            pl.BlockSpec((TROWS, K), lambda i: (i, 0),
                         memory_space=pltpu.SMEM),
            pl.BlockSpec((N, M), lambda i: (0, 0)),
        ],
        out_specs=pl.BlockSpec((TROWS, M), lambda i: (i, 0)),
        out_shape=jax.ShapeDtypeStruct((TCN, M), jnp.float32),
    )(idx_tc, x)


def _tc_matmul_body(x_ref, hnb_ref, wt_ref, b_ref, y_ref, s_ref, s2_ref):
    i = pl.program_id(0)
    h = x_ref[...] + hnb_ref[...]
    y = jnp.dot(h, wt_ref[...], preferred_element_type=jnp.float32) + b_ref[...]
    y_ref[...] = y
    s = jnp.sum(y, axis=0, keepdims=True)
    s2 = jnp.sum(y * y, axis=0, keepdims=True)

    @pl.when(i == 0)
    def _():
        s_ref[...] = s
        s2_ref[...] = s2

    @pl.when(i > 0)
    def _():
        s_ref[...] += s
        s2_ref[...] += s2


def _tc_bn_body(y_ref, s_ref, s2_ref, g_ref, beta_ref, o_ref):
    mean = s_ref[...] * (1.0 / N)
    var = s2_ref[...] * (1.0 / N) - mean * mean
    scale = g_ref[...] * lax.rsqrt(var + 1e-5)
    shift = beta_ref[...] - mean * scale
    o_ref[...] = jnp.maximum(y_ref[...] * scale + shift, 0.0)


def kernel(nodes_features, nodes_neighbors_indexes, W, b, gamma, beta):
    x = nodes_features
    idxf_pad = jnp.pad(nodes_neighbors_indexes.reshape(-1),
                       (0, (N_PAD - N) * K))
    hnb_sc = _neighbor_sum(x, idxf_pad[TCN * K:])
    hnb_tc = _tc_gather(x, nodes_neighbors_indexes[:TCN])
    hnb = jnp.concatenate([hnb_tc, hnb_sc], axis=0)

    wt = W.T
    b2 = b.reshape(1, OUT)
    g2 = gamma.reshape(1, OUT)
    beta2 = beta.reshape(1, OUT)

    y, s, s2 = pl.pallas_call(
        _tc_matmul_body,
        grid=(GRID,),
        in_specs=[
            pl.BlockSpec((ROWS, M), lambda i: (i, 0)),
            pl.BlockSpec((ROWS, M), lambda i: (i, 0)),
            pl.BlockSpec((M, OUT), lambda i: (0, 0)),
            pl.BlockSpec((1, OUT), lambda i: (0, 0)),
        ],
        out_specs=[
            pl.BlockSpec((ROWS, OUT), lambda i: (i, 0)),
            pl.BlockSpec((1, OUT), lambda i: (0, 0)),
            pl.BlockSpec((1, OUT), lambda i: (0, 0)),
        ],
        out_shape=[
            jax.ShapeDtypeStruct((N, OUT), jnp.float32),
            jax.ShapeDtypeStruct((1, OUT), jnp.float32),
            jax.ShapeDtypeStruct((1, OUT), jnp.float32),
        ],
    )(x, hnb, wt, b2)

    out = pl.pallas_call(
        _tc_bn_body,
        grid=(GRID,),
        in_specs=[
            pl.BlockSpec((ROWS, OUT), lambda i: (i, 0)),
            pl.BlockSpec((1, OUT), lambda i: (0, 0)),
            pl.BlockSpec((1, OUT), lambda i: (0, 0)),
            pl.BlockSpec((1, OUT), lambda i: (0, 0)),
            pl.BlockSpec((1, OUT), lambda i: (0, 0)),
        ],
        out_specs=pl.BlockSpec((ROWS, OUT), lambda i: (i, 0)),
        out_shape=jax.ShapeDtypeStruct((N, OUT), jnp.float32),
    )(y, s, s2, g2, beta2)

    return (out, nodes_neighbors_indexes)


# speedup vs baseline: 1.1786x; 1.0716x over previous
import functools

import jax
import jax.numpy as jnp
from jax import lax
from jax.experimental import pallas as pl
from jax.experimental.pallas import tpu as pltpu
from jax.experimental.pallas import tpu_sc as plsc

N = 10000
M = 256
K = 16
OUT = 512

NC, NS = 2, 16
NW = NC * NS
DV = M // 16


C3 = 8

FAST_CORE = 0
MP = 128
GP = 8
TCN = 4000
TROWS = 200
FW, SW = 304, 80
FAST_TOT = NS * FW
N_PAD_SC = NS * (FW + SW)
N_PAD = TCN + N_PAD_SC


NBUF = 2


def _sc_body(x_hbm, idxf_hbm, h_hbm,
             idx_all, buf0, buf1, out_v,
             gsem0, gsem1, wsem):
    c = lax.axis_index("c")
    s = lax.axis_index("s")
    is_fast = c == FAST_CORE
    base = jnp.where(is_fast, s * FW, FAST_TOT + s * SW)
    nchunk = jnp.where(is_fast, FW // C3, SW // C3)
    nrounds = nchunk // NBUF
    bufs = (buf0, buf1)
    gsems = (gsem0, gsem1)

    @pl.when(is_fast)
    def _():
        pltpu.sync_copy(idxf_hbm.at[pl.ds(base * K, FW * K)], idx_all)

    @pl.when(jnp.logical_not(is_fast))
    def _():
        pltpu.sync_copy(idxf_hbm.at[pl.ds(base * K, SW * K)],
                        idx_all.at[pl.ds(0, SW * K)])

    def gdesc(chunk, b):
        return pltpu.make_async_copy(
            x_hbm.at[idx_all.at[pl.ds(chunk * (C3 * K), C3 * K)]],
            bufs[b], gsems[b])

    for bb in range(NBUF):
        gdesc(bb, bb).start()

    @pl.loop(0, nrounds)
    def _round(g):
        @pl.when(g > 0)
        def _():
            pltpu.make_async_copy(out_v, h_hbm.at[pl.ds(0, NBUF * C3)],
                                  wsem).wait()
        for b in range(NBUF):
            chunk = NBUF * g + b
            buf = bufs[b]
            gdesc(chunk, b).wait()

            @plsc.parallel_loop(0, C3 * DV)
            def _red(i):
                cc = i // DV
                sl = pl.ds((i % DV) * 16, 16)
                rb = cc * K
                vals = [buf[rb + k, sl] for k in range(K)]
                while len(vals) > 1:
                    vals = [vals[j] + vals[j + 1]
                            for j in range(0, len(vals) - 1, 2)] + (
                                [vals[-1]] if len(vals) % 2 else [])
                out_v[b * C3 + cc, sl] = vals[0]

            @pl.when(chunk + NBUF < nchunk)
            def _():
                gdesc(chunk + NBUF, b).start()

        pltpu.async_copy(
            out_v, h_hbm.at[pl.ds(base + g * (NBUF * C3), NBUF * C3)], wsem)

    pltpu.make_async_copy(out_v, h_hbm.at[pl.ds(0, NBUF * C3)], wsem).wait()


def _neighbor_sum(x, idxf_pad):
    kfn = pl.kernel(
        _sc_body,
        out_type=jax.ShapeDtypeStruct((N_PAD_SC, M), jnp.float32),
        mesh=plsc.VectorSubcoreMesh(core_axis_name="c", subcore_axis_name="s"),
        scratch_types=[
            pltpu.VMEM((FW * K,), jnp.int32),
            pltpu.VMEM((C3 * K, M), jnp.float32),
            pltpu.VMEM((C3 * K, M), jnp.float32),
            pltpu.VMEM((NBUF * C3, M), jnp.float32),
            pltpu.SemaphoreType.DMA,
            pltpu.SemaphoreType.DMA,
            pltpu.SemaphoreType.DMA,
        ],
    )
    return kfn(x, idxf_pad)


ROWS = 1000
GRID = N // ROWS


def _tc_gather_body(idx_ref, x_ref, o_ref):
    def body(n, carry):
        acc = x_ref[pl.ds(idx_ref[n, 0], 1), :]
        for k in range(1, K):
            acc = acc + x_ref[pl.ds(idx_ref[n, k], 1), :]
        o_ref[pl.ds(n, 1), :] = acc
        return carry

    lax.fori_loop(0, TROWS, body, 0)


def _tc_gather(x, idx_tc):
    return pl.pallas_call(
        _tc_gather_body,
        grid=(TCN // TROWS,),
        in_specs=[
            pl.BlockSpec((TROWS, K), lambda i: (i, 0),
                         memory_space=pltpu.SMEM),
            pl.BlockSpec((N, M), lambda i: (0, 0)),
        ],
        out_specs=pl.BlockSpec((TROWS, M), lambda i: (i, 0)),
        out_shape=jax.ShapeDtypeStruct((TCN, M), jnp.float32),
    )(idx_tc, x)


def _tc_matmul_body(x_ref, hnb_ref, wt_ref, b_ref, y_ref, s_ref, s2_ref):
    i = pl.program_id(0)
    h = x_ref[...] + hnb_ref[...]
    y = jnp.dot(h, wt_ref[...], preferred_element_type=jnp.float32) + b_ref[...]
    y_ref[...] = y
    s = jnp.sum(y, axis=0, keepdims=True)
    s2 = jnp.sum(y * y, axis=0, keepdims=True)

    @pl.when(i == 0)
    def _():
        s_ref[...] = s
        s2_ref[...] = s2

    @pl.when(i > 0)
    def _():
        s_ref[...] += s
        s2_ref[...] += s2


def _tc_bn_body(y_ref, s_ref, s2_ref, g_ref, beta_ref, o_ref):
    mean = s_ref[...] * (1.0 / N)
    var = s2_ref[...] * (1.0 / N) - mean * mean
    scale = g_ref[...] * lax.rsqrt(var + 1e-5)
    shift = beta_ref[...] - mean * scale
    o_ref[...] = jnp.maximum(y_ref[...] * scale + shift, 0.0)


def kernel(nodes_features, nodes_neighbors_indexes, W, b, gamma, beta):
    x = nodes_features
    idxf_pad = jnp.pad(nodes_neighbors_indexes.reshape(-1),
                       (0, (N_PAD - N) * K))
    hnb_sc = _neighbor_sum(x, idxf_pad[TCN * K:])
    hnb_tc = _tc_gather(x, nodes_neighbors_indexes[:TCN])
    hnb = jnp.concatenate([hnb_tc, hnb_sc], axis=0)

    wt = W.T
    b2 = b.reshape(1, OUT)
    g2 = gamma.reshape(1, OUT)
    beta2 = beta.reshape(1, OUT)

    y, s, s2 = pl.pallas_call(
        _tc_matmul_body,
        grid=(GRID,),
        in_specs=[
            pl.BlockSpec((ROWS, M), lambda i: (i, 0)),
            pl.BlockSpec((ROWS, M), lambda i: (i, 0)),
            pl.BlockSpec((M, OUT), lambda i: (0, 0)),
            pl.BlockSpec((1, OUT), lambda i: (0, 0)),
        ],
        out_specs=[
            pl.BlockSpec((ROWS, OUT), lambda i: (i, 0)),
            pl.BlockSpec((1, OUT), lambda i: (0, 0)),
            pl.BlockSpec((1, OUT), lambda i: (0, 0)),
        ],
        out_shape=[
            jax.ShapeDtypeStruct((N, OUT), jnp.float32),
            jax.ShapeDtypeStruct((1, OUT), jnp.float32),
            jax.ShapeDtypeStruct((1, OUT), jnp.float32),
        ],
    )(x, hnb, wt, b2)

    out = pl.pallas_call(
        _tc_bn_body,
        grid=(GRID,),
        in_specs=[
            pl.BlockSpec((ROWS, OUT), lambda i: (i, 0)),
            pl.BlockSpec((1, OUT), lambda i: (0, 0)),
            pl.BlockSpec((1, OUT), lambda i: (0, 0)),
            pl.BlockSpec((1, OUT), lambda i: (0, 0)),
            pl.BlockSpec((1, OUT), lambda i: (0, 0)),
        ],
        out_specs=pl.BlockSpec((ROWS, OUT), lambda i: (i, 0)),
        out_shape=jax.ShapeDtypeStruct((N, OUT), jnp.float32),
    )(y, s, s2, g2, beta2)

    return (out, nodes_neighbors_indexes)


# speedup vs baseline: 1.2177x; 1.0332x over previous
import functools

import jax
import jax.numpy as jnp
from jax import lax
from jax.experimental import pallas as pl
from jax.experimental.pallas import tpu as pltpu
from jax.experimental.pallas import tpu_sc as plsc

N = 10000
M = 256
K = 16
OUT = 512

NC, NS = 2, 16
NW = NC * NS
DV = M // 16


C3 = 8

FAST_CORE = 0
MP = 128
GP = 8
TCN = 4000
TROWS = 200
FW, SW = 304, 80
FAST_TOT = NS * FW
N_PAD_SC = NS * (FW + SW)
N_PAD = TCN + N_PAD_SC


NBUF = 2


def _sc_body(x_hbm, idxf_hbm, h_hbm,
             idx_all, buf0, buf1, out_v,
             gsem0, gsem1, wsem):
    c = lax.axis_index("c")
    s = lax.axis_index("s")
    is_fast = c == FAST_CORE
    base = jnp.where(is_fast, s * FW, FAST_TOT + s * SW)
    nchunk = jnp.where(is_fast, FW // C3, SW // C3)
    nrounds = nchunk // NBUF
    bufs = (buf0, buf1)
    gsems = (gsem0, gsem1)

    @pl.when(is_fast)
    def _():
        pltpu.sync_copy(idxf_hbm.at[pl.ds(base * K, FW * K)], idx_all)

    @pl.when(jnp.logical_not(is_fast))
    def _():
        pltpu.sync_copy(idxf_hbm.at[pl.ds(base * K, SW * K)],
                        idx_all.at[pl.ds(0, SW * K)])

    def gdesc(chunk, b):
        return pltpu.make_async_copy(
            x_hbm.at[idx_all.at[pl.ds(chunk * (C3 * K), C3 * K)]],
            bufs[b], gsems[b])

    for bb in range(NBUF):
        gdesc(bb, bb).start()

    @pl.loop(0, nrounds)
    def _round(g):
        @pl.when(g > 0)
        def _():
            pltpu.make_async_copy(out_v, h_hbm.at[pl.ds(0, NBUF * C3)],
                                  wsem).wait()
        for b in range(NBUF):
            chunk = NBUF * g + b
            buf = bufs[b]
            gdesc(chunk, b).wait()

            @plsc.parallel_loop(0, C3 * DV)
            def _red(i):
                cc = i // DV
                sl = pl.ds((i % DV) * 16, 16)
                rb = cc * K
                vals = [buf[rb + k, sl] for k in range(K)]
                while len(vals) > 1:
                    vals = [vals[j] + vals[j + 1]
                            for j in range(0, len(vals) - 1, 2)] + (
                                [vals[-1]] if len(vals) % 2 else [])
                out_v[b * C3 + cc, sl] = vals[0]

            @pl.when(chunk + NBUF < nchunk)
            def _():
                gdesc(chunk + NBUF, b).start()

        pltpu.async_copy(
            out_v, h_hbm.at[pl.ds(base + g * (NBUF * C3), NBUF * C3)], wsem)

    pltpu.make_async_copy(out_v, h_hbm.at[pl.ds(0, NBUF * C3)], wsem).wait()


def _neighbor_sum(x, idxf_pad):
    kfn = pl.kernel(
        _sc_body,
        out_type=jax.ShapeDtypeStruct((N_PAD_SC, M), jnp.float32),
        mesh=plsc.VectorSubcoreMesh(core_axis_name="c", subcore_axis_name="s"),
        scratch_types=[
            pltpu.VMEM((FW * K,), jnp.int32),
            pltpu.VMEM((C3 * K, M), jnp.float32),
            pltpu.VMEM((C3 * K, M), jnp.float32),
            pltpu.VMEM((NBUF * C3, M), jnp.float32),
            pltpu.SemaphoreType.DMA,
            pltpu.SemaphoreType.DMA,
            pltpu.SemaphoreType.DMA,
        ],
    )
    return kfn(x, idxf_pad)


ROWS = 1000
GRID = N // ROWS


def _tc_gather_body(idx_ref, x_ref, o_ref):
    def body(n, carry):
        acc = x_ref[pl.ds(idx_ref[n, 0], 1), :]
        for k in range(1, K):
            acc = acc + x_ref[pl.ds(idx_ref[n, k], 1), :]
        o_ref[pl.ds(n, 1), :] = acc
        return carry

    lax.fori_loop(0, TROWS, body, 0)


def _tc_gather(x, idx_tc):
    return pl.pallas_call(
        _tc_gather_body,
        grid=(TCN // TROWS,),
        in_specs=[
            pl.BlockSpec((TROWS, K), lambda i: (i, 0),
                         memory_space=pltpu.SMEM),
            pl.BlockSpec((N, M), lambda i: (0, 0)),
        ],
        out_specs=pl.BlockSpec((TROWS, M), lambda i: (i, 0)),
        out_shape=jax.ShapeDtypeStruct((TCN, M), jnp.float32),
    )(idx_tc, x)


TCB = TCN // 1000


def _tc_matmul_body(x_ref, tc_ref, sc_ref, wt_ref, b_ref,
                    y_ref, s_ref, s2_ref, h_scr):
    i = pl.program_id(0)

    @pl.when(i < TCB)
    def _():
        h_scr[...] = x_ref[...] + tc_ref[...]

    @pl.when(i >= TCB)
    def _():
        h_scr[...] = x_ref[...] + sc_ref[...]

    h = h_scr[...]
    y = jnp.dot(h, wt_ref[...], preferred_element_type=jnp.float32) + b_ref[...]
    y_ref[...] = y
    s = jnp.sum(y, axis=0, keepdims=True)
    s2 = jnp.sum(y * y, axis=0, keepdims=True)

    @pl.when(i == 0)
    def _():
        s_ref[...] = s
        s2_ref[...] = s2

    @pl.when(i > 0)
    def _():
        s_ref[...] += s
        s2_ref[...] += s2


def _tc_bn_body(y_ref, s_ref, s2_ref, g_ref, beta_ref, o_ref):
    mean = s_ref[...] * (1.0 / N)
    var = s2_ref[...] * (1.0 / N) - mean * mean
    scale = g_ref[...] * lax.rsqrt(var + 1e-5)
    shift = beta_ref[...] - mean * scale
    o_ref[...] = jnp.maximum(y_ref[...] * scale + shift, 0.0)


def kernel(nodes_features, nodes_neighbors_indexes, W, b, gamma, beta):
    x = nodes_features
    idxf_pad = jnp.pad(nodes_neighbors_indexes.reshape(-1),
                       (0, (N_PAD - N) * K))
    hnb_sc = _neighbor_sum(x, idxf_pad[TCN * K:])
    hnb_tc = _tc_gather(x, nodes_neighbors_indexes[:TCN])

    wt = W.T
    b2 = b.reshape(1, OUT)
    g2 = gamma.reshape(1, OUT)
    beta2 = beta.reshape(1, OUT)

    y, s, s2 = pl.pallas_call(
        _tc_matmul_body,
        grid=(GRID,),
        in_specs=[
            pl.BlockSpec((ROWS, M), lambda i: (i, 0)),
            pl.BlockSpec((ROWS, M), lambda i: (jnp.minimum(i, TCB - 1), 0)),
            pl.BlockSpec((ROWS, M), lambda i: (jnp.maximum(i - TCB, 0), 0)),
            pl.BlockSpec((M, OUT), lambda i: (0, 0)),
            pl.BlockSpec((1, OUT), lambda i: (0, 0)),
        ],
        scratch_shapes=[pltpu.VMEM((ROWS, M), jnp.float32)],
        out_specs=[
            pl.BlockSpec((ROWS, OUT), lambda i: (i, 0)),
            pl.BlockSpec((1, OUT), lambda i: (0, 0)),
            pl.BlockSpec((1, OUT), lambda i: (0, 0)),
        ],
        out_shape=[
            jax.ShapeDtypeStruct((N, OUT), jnp.float32),
            jax.ShapeDtypeStruct((1, OUT), jnp.float32),
            jax.ShapeDtypeStruct((1, OUT), jnp.float32),
        ],
    )(x, hnb_tc, hnb_sc, wt, b2)

    out = pl.pallas_call(
        _tc_bn_body,
        grid=(GRID,),
        in_specs=[
            pl.BlockSpec((ROWS, OUT), lambda i: (i, 0)),
            pl.BlockSpec((1, OUT), lambda i: (0, 0)),
            pl.BlockSpec((1, OUT), lambda i: (0, 0)),
            pl.BlockSpec((1, OUT), lambda i: (0, 0)),
            pl.BlockSpec((1, OUT), lambda i: (0, 0)),
        ],
        out_specs=pl.BlockSpec((ROWS, OUT), lambda i: (i, 0)),
        out_shape=jax.ShapeDtypeStruct((N, OUT), jnp.float32),
    )(y, s, s2, g2, beta2)

    return (out, nodes_neighbors_indexes)
